# baseline (speedup 1.0000x reference)
import jax
import jax.numpy as jnp
from jax import lax
from jax.experimental import pallas as pl
from jax.experimental.pallas import tpu as pltpu
from jax.experimental.pallas import tpu_sc as plsc

B = 64
N = 10000
F = 64
T = 288
TP = 289

NC, NS, L = 2, 16, 16
NW = NC * NS
BPW = B // NW
NB = 400
NBP = 401
CH = N // NB
GPC = NB // L
TPW = BPW * CH
FB = F // L


def _body(xs_hbm, tab_hbm, out_hbm, tab_v, xs_v, ob, sems):
    wid = lax.axis_index("s") * NC + lax.axis_index("c")
    b0 = wid * BPW
    pltpu.sync_copy(tab_hbm, tab_v)
    for i in range(BPW):
        pltpu.sync_copy(xs_hbm.at[b0 + i], xs_v.at[i])

    lane = lax.iota(jnp.int32, L)
    g_addr = [(lane + k * L) * TP for k in range(FB)]
    s_idx = [lane + k * L for k in range(FB)]

    def pair(kk, carry):
        for j in range(2):
            t = kk * 2 + j
            bl = t // CH
            c = t % CH
            dst = out_hbm.at[b0 + bl, :, pl.ds(c * NB, NB)]

            @pl.when(kk > 0)
            def _wait_prev():
                pltpu.make_async_copy(ob[j].at[:, pl.ds(0, NB)], dst,
                                      sems[j]).wait()

            @plsc.parallel_loop(0, GPC, unroll=2)
            def per_group(g):
                xv = xs_v[bl, pl.ds(c * NB + g * L, L)]
                tvec = jnp.clip((xv * 288.0).astype(jnp.int32), 0, T - 1)
                for i in range(L):
                    tn = jnp.broadcast_to(tvec[i], (L,))
                    ns = jnp.broadcast_to(g * L + i, (L,))
                    for k in range(FB):
                        vals = plsc.load_gather(tab_v, [g_addr[k] + tn])
                        plsc.store_scatter(ob[j], [s_idx[k], ns], vals)

            pltpu.async_copy(ob[j].at[:, pl.ds(0, NB)], dst, sems[j])
        return carry

    lax.fori_loop(0, TPW // 2, pair, 0)
    for j in range(2):
        dst = out_hbm.at[b0, :, pl.ds(0, NB)]
        pltpu.make_async_copy(ob[j].at[:, pl.ds(0, NB)], dst, sems[j]).wait()


_sc = pl.kernel(
    _body,
    out_type=jax.ShapeDtypeStruct((B, F, N), jnp.float32),
    mesh=plsc.VectorSubcoreMesh(
        core_axis_name="c", subcore_axis_name="s",
        num_cores=NC, num_subcores=NS,
    ),
    scratch_types=[
        pltpu.VMEM((F * TP,), jnp.float32),
        pltpu.VMEM((BPW, N), jnp.float32),
        [pltpu.VMEM((F, NBP), jnp.float32)] * 2,
        [pltpu.SemaphoreType.DMA] * 2,
    ],
    compiler_params=pltpu.CompilerParams(
        use_tc_tiling_on_sc=False, needs_layout_passes=False,
        disable_bounds_checks=True,
    ),
)


def kernel(x, time_day):
    xs = x[:, -1, :, 1]
    tab_t = jnp.pad(time_day.T, ((0, 0), (0, 1))).reshape(F * TP)
    return _sc(xs, tab_t)[..., None]


# speedup vs baseline: 1.1430x; 1.1430x over previous
import jax
import jax.numpy as jnp
from jax import lax
from jax.experimental import pallas as pl
from jax.experimental.pallas import tpu as pltpu
from jax.experimental.pallas import tpu_sc as plsc

B = 64
N = 10000
F = 64
T = 288

NC, NS, L = 2, 16, 16
NW = NC * NS
BPW = B // NW
NB = 400
CH = N // NB
GPC = NB // L
TPW = BPW * CH


def _body(xs_hbm, tab_hbm, out_hbm, tab_v, xs_v, ob, sems):
    wid = lax.axis_index("s") * NC + lax.axis_index("c")
    b0 = wid * BPW
    pltpu.sync_copy(tab_hbm, tab_v)
    for i in range(BPW):
        pltpu.sync_copy(xs_hbm.at[b0 + i], xs_v.at[i])

    def pair(kk, carry):
        for j in range(2):
            t = kk * 2 + j
            bl = t // CH
            c = t % CH
            dst = out_hbm.at[b0 + bl, :, pl.ds(c * NB, NB)]

            @pl.when(kk > 0)
            def _wait_prev():
                pltpu.make_async_copy(ob[j], dst, sems[j]).wait()

            @plsc.parallel_loop(0, GPC, unroll=5)
            def per_group(g):
                xv = xs_v[bl, pl.ds(c * NB + g * L, L)]
                tt = jnp.clip((xv * 288.0).astype(jnp.int32), 0, T - 1)
                for f in range(F):
                    vals = plsc.load_gather(tab_v, [tt + f * T])
                    ob[j][f, pl.ds(g * L, L)] = vals

            pltpu.async_copy(ob[j], dst, sems[j])
        return carry

    lax.fori_loop(0, TPW // 2, pair, 0)
    for j in range(2):
        dst = out_hbm.at[b0, :, pl.ds(0, NB)]
        pltpu.make_async_copy(ob[j], dst, sems[j]).wait()


_sc = pl.kernel(
    _body,
    out_type=jax.ShapeDtypeStruct((B, F, N), jnp.float32),
    mesh=plsc.VectorSubcoreMesh(
        core_axis_name="c", subcore_axis_name="s",
        num_cores=NC, num_subcores=NS,
    ),
    scratch_types=[
        pltpu.VMEM((F * T,), jnp.float32),
        pltpu.VMEM((BPW, N), jnp.float32),
        [pltpu.VMEM((F, NB), jnp.float32)] * 2,
        [pltpu.SemaphoreType.DMA] * 2,
    ],
    compiler_params=pltpu.CompilerParams(
        use_tc_tiling_on_sc=False, needs_layout_passes=False,
        disable_bounds_checks=True,
    ),
)


def kernel(x, time_day):
    xs = x[:, -1, :, 1]
    tab_t = time_day.T.reshape(F * T)
    return _sc(xs, tab_t)[..., None]


# speedup vs baseline: 1.2943x; 1.1324x over previous
import jax
import jax.numpy as jnp
from jax import lax
from jax.experimental import pallas as pl
from jax.experimental.pallas import tpu as pltpu
from jax.experimental.pallas import tpu_sc as plsc

B = 64
N = 10000
F = 64
T = 288

NC, NS, L = 2, 16, 16
NW = NC * NS
BPW = B // NW
FPD = 4
DPB = F // FPD
TPW = BPW * DPB
GPB = N // L


def _body(xs_hbm, tab_hbm, out_hbm, tab_v, xs_v, tt_v, ob, sems):
    wid = lax.axis_index("s") * NC + lax.axis_index("c")
    b0 = wid * BPW
    pltpu.sync_copy(tab_hbm, tab_v)

    def pair(kk, carry):
        for j in range(2):
            t = kk * 2 + j
            bl = t // DPB
            fb = t % DPB
            start = ((b0 + bl) * F + fb * FPD) * N
            dst = out_hbm.at[pl.ds(start, FPD * N)]

            @pl.when(fb == 0)
            def _stage_batch():
                pltpu.sync_copy(xs_hbm.at[b0 + bl], xs_v)

                @plsc.parallel_loop(0, GPB, unroll=5)
                def pre(g):
                    xv = xs_v[pl.ds(g * L, L)]
                    tt_v[pl.ds(g * L, L)] = jnp.clip(
                        (xv * 288.0).astype(jnp.int32), 0, T - 1)

            @pl.when(kk > 0)
            def _wait_prev():
                pltpu.make_async_copy(ob[j], dst, sems[j]).wait()

            f0 = fb * FPD

            @plsc.parallel_loop(0, GPB, unroll=5)
            def per_group(g):
                tvec = tt_v[pl.ds(g * L, L)]
                for fl in range(FPD):
                    vals = plsc.load_gather(tab_v, [tvec + (f0 + fl) * T])
                    ob[j][pl.ds(fl * N + g * L, L)] = vals

            pltpu.async_copy(ob[j], dst, sems[j])
        return carry

    lax.fori_loop(0, TPW // 2, pair, 0)
    for j in range(2):
        dst = out_hbm.at[pl.ds(b0 * F * N, FPD * N)]
        pltpu.make_async_copy(ob[j], dst, sems[j]).wait()


_sc = pl.kernel(
    _body,
    out_type=jax.ShapeDtypeStruct((B * F * N,), jnp.float32),
    mesh=plsc.VectorSubcoreMesh(
        core_axis_name="c", subcore_axis_name="s",
        num_cores=NC, num_subcores=NS,
    ),
    scratch_types=[
        pltpu.VMEM((F * T,), jnp.float32),
        pltpu.VMEM((N,), jnp.float32),
        pltpu.VMEM((N,), jnp.int32),
        [pltpu.VMEM((FPD * N,), jnp.float32)] * 2,
        [pltpu.SemaphoreType.DMA] * 2,
    ],
    compiler_params=pltpu.CompilerParams(
        use_tc_tiling_on_sc=False, needs_layout_passes=False,
        disable_bounds_checks=True,
    ),
)


def kernel(x, time_day):
    xs = x[:, -1, :, 1]
    tab_t = time_day.T.reshape(F * T)
    return _sc(xs, tab_t).reshape(B, F, N, 1)
